# Initial kernel scaffold; baseline (speedup 1.0000x reference)
import jax
import jax.numpy as jnp
from jax.experimental import pallas as pl


def kernel(preds, targets):
    raise NotImplementedError("write your pallas kernel here")


# speedup vs baseline: 109.3882x; 109.3882x over previous
import jax
import jax.numpy as jnp
from jax import lax
from jax.experimental import pallas as pl

_N = 4096
_R = 256
_G = _N // _R


def _tc_body(x_ref, tgt_ref, out_ref):
    i = pl.program_id(0)
    x = x_ref[...]
    col = lax.broadcasted_iota(jnp.int32, (_R, _N), 1)
    neg = jnp.float32(-jnp.inf)
    m0 = jnp.max(x, axis=1, keepdims=True)
    m1 = jnp.max(jnp.where(x < m0, x, neg), axis=1, keepdims=True)
    m2 = jnp.max(jnp.where(x < m1, x, neg), axis=1, keepdims=True)
    s = jnp.sum(jnp.exp(x - m0), axis=1, keepdims=True)
    lse = m0 + jnp.log(s)
    rowid = i * _R + lax.broadcasted_iota(jnp.int32, (_R, 1), 0)
    tb = tgt_ref[...]
    d = jnp.sum(jnp.where(col == rowid, x, 0.0), axis=1, keepdims=True)
    t = jnp.sum(jnp.where(col == tb, x, 0.0), axis=1, keepdims=True)
    is0 = d == m0
    is1 = d == m1
    va = jnp.where(is0, m1, m0)
    vb = jnp.where(is0 | is1, m2, m1)
    ind = ((tb != rowid) & (t != va) & (t != vb)).astype(jnp.float32)
    loss = lse * (1.0 + ind) - (0.7 * d + 0.2 * va + 0.1 * vb + ind * t)
    part = jnp.sum(loss, axis=0, keepdims=True) * jnp.float32(1.0 / _N)
    prev = jnp.where(i == 0, jnp.zeros_like(part), out_ref[...])
    out_ref[...] = prev + part


def kernel(preds, targets):
    tgt = targets.astype(jnp.int32).reshape(_N, 1)
    out = pl.pallas_call(
        _tc_body,
        grid=(_G,),
        in_specs=[
            pl.BlockSpec((_R, _N), lambda i: (i, 0)),
            pl.BlockSpec((_R, 1), lambda i: (i, 0)),
        ],
        out_specs=pl.BlockSpec((1, 1), lambda i: (0, 0)),
        out_shape=jax.ShapeDtypeStruct((1, 1), jnp.float32),
    )(preds, tgt)
    return out[0, 0]
